# baseline (speedup 1.0000x reference)
import jax
import jax.numpy as jnp
from jax import lax
from jax.experimental import pallas as pl
from jax.experimental.pallas import tpu as pltpu

_N = 16384
_C = 512
_K = 1024
_BR = 2048
_NB = _N // _BR
_ROWS = 16
_COLS = _N // _ROWS
_EPS = 1e-8
_HI = lax.Precision.HIGHEST


def _sortable_u32(x):
    u = lax.bitcast_convert_type(x, jnp.uint32)
    sign = (u >> jnp.uint32(31)).astype(jnp.uint32)
    flip = jnp.where(sign == jnp.uint32(1),
                     jnp.uint32(0xFFFFFFFF), jnp.uint32(0x80000000))
    return u ^ flip


def _kth_threshold(keys_p, keys_n, k):
    kf = jnp.float32(k)

    def body(_, carry):
        lo_p, hi_p, lo_n, hi_n = carry
        mid_p = lo_p + ((hi_p - lo_p) >> jnp.uint32(1)) + ((hi_p - lo_p) & jnp.uint32(1))
        mid_n = lo_n + ((hi_n - lo_n) >> jnp.uint32(1)) + ((hi_n - lo_n) & jnp.uint32(1))
        cnt_p = jnp.sum(jnp.where(keys_p >= mid_p, 1.0, 0.0))
        cnt_n = jnp.sum(jnp.where(keys_n >= mid_n, 1.0, 0.0))
        ok_p = cnt_p >= kf
        ok_n = cnt_n >= kf
        lo_p = jnp.where(ok_p, mid_p, lo_p)
        hi_p = jnp.where(ok_p, hi_p, mid_p - jnp.uint32(1))
        lo_n = jnp.where(ok_n, mid_n, lo_n)
        hi_n = jnp.where(ok_n, hi_n, mid_n - jnp.uint32(1))
        return lo_p, hi_p, lo_n, hi_n

    z = jnp.uint32(0)
    m = jnp.uint32(0xFFFFFFFF)
    lo_p, _, lo_n, _ = lax.fori_loop(0, 32, body, (z, m, z, m))
    return lo_p, lo_n


def _flat_ranks(eq_f32):
    ci = lax.broadcasted_iota(jnp.int32, (_COLS, _COLS), 0)
    cj = lax.broadcasted_iota(jnp.int32, (_COLS, _COLS), 1)
    upper = (ci < cj).astype(jnp.float32)
    row_pfx = jnp.dot(eq_f32, upper)
    ri = lax.broadcasted_iota(jnp.int32, (_ROWS, _ROWS), 0)
    rj = lax.broadcasted_iota(jnp.int32, (_ROWS, _ROWS), 1)
    lower = (rj < ri).astype(jnp.float32)
    row_sums = jnp.sum(eq_f32, axis=1, keepdims=True)
    offs = jnp.dot(lower, row_sums, precision=_HI)
    return row_pfx + offs


def _select_mask(keys, thresh, k):
    gt = (keys > thresh).astype(jnp.float32)
    eq = (keys == thresh).astype(jnp.float32)
    cnt_gt = jnp.sum(gt)
    r = jnp.float32(k) - cnt_gt
    ranks = _flat_ranks(eq)
    pick_eq = eq * jnp.where(ranks < r, 1.0, 0.0)
    return gt + pick_eq


def _dense_body(rhs3_ref, x_ref, dm_ref):
    xb = x_ref[...].astype(jnp.bfloat16)
    rhs3b = rhs3_ref[...].astype(jnp.bfloat16)
    lhs2 = jnp.concatenate([xb, xb * xb], axis=1)
    dm_ref[...] = jnp.dot(lhs2, rhs3b, preferred_element_type=jnp.float32)


def _loss_body(pn_ref, lg_ref, mk_ref, gp_ref, gn_ref, d0_ref, d1_ref,
               sq_ref, out_ref):
    lg = lg_ref[...]
    mk = mk_ref[...]
    sar = jnp.abs(mk - lg)
    w3 = sar * sar * sar
    wp = w3 * mk
    wn = w3 * (1.0 - mk)
    neg_inf = jnp.float32(-jnp.inf)
    logw_p = jnp.where(wp > 0, jnp.log(jnp.maximum(wp, 1e-30)), neg_inf)
    logw_n = jnp.where(wn > 0, jnp.log(jnp.maximum(wn, 1e-30)), neg_inf)
    score_p = logw_p + gp_ref[...]
    score_n = logw_n + gn_ref[...]
    keys_p = _sortable_u32(score_p)
    keys_n = _sortable_u32(score_n)
    t_p, t_n = _kth_threshold(keys_p, keys_n, _K)
    sel_p = _select_mask(keys_p, t_p, _K)
    sel_n = _select_mask(keys_n, t_n, _K)

    pn = pn_ref[...]
    d0 = d0_ref[...]
    d1 = d1_ref[...]
    sq_all = sq_ref[...]
    na = jnp.maximum(jnp.sqrt(sq_all), _EPS)
    nb0 = jnp.maximum(jnp.sqrt(jnp.sum(pn[:, 0:1] * pn[:, 0:1])), _EPS)
    nb1 = jnp.maximum(jnp.sqrt(jnp.sum(pn[:, 1:2] * pn[:, 1:2])), _EPS)
    s0 = d0 / (na * nb0)
    s1 = d1 / (na * nb1)
    m = jnp.maximum(s0, s1)
    lse = m + jnp.log(jnp.exp(s0 - m) + jnp.exp(s1 - m))
    l0 = lse - s0
    l1 = lse - s1

    sum_p = jnp.sum(sel_p * l0)
    sum_n = jnp.sum(sel_n * l1)
    loss = (sum_p + sum_n) / jnp.float32(2 * _K)
    out_ref[...] = jnp.reshape(loss, (1, 1))


def kernel(inputs, logits, protos, mask):
    skey = jax.random.key(42)
    kp, kn = jax.random.split(skey)
    g_pos = jax.random.gumbel(kp, (_N,), jnp.float32).reshape(_ROWS, _COLS)
    g_neg = jax.random.gumbel(kn, (_N,), jnp.float32).reshape(_ROWS, _COLS)
    p2t = jnp.transpose(protos[:, :, 0])
    rhs3 = jnp.zeros((2 * _C, 3), jnp.float32)
    rhs3 = rhs3.at[0:_C, 0:2].set(p2t)
    rhs3 = rhs3.at[_C:2 * _C, 2].set(1.0)
    lg = logits.reshape(_ROWS, _COLS)
    mk = mask.reshape(_ROWS, _COLS)

    dm3 = pl.pallas_call(
        _dense_body,
        grid=(_NB,),
        in_specs=[
            pl.BlockSpec((2 * _C, 3), lambda i: (0, 0)),
            pl.BlockSpec((_BR, _C), lambda i: (i, 0)),
        ],
        out_specs=pl.BlockSpec((_BR, 3), lambda i: (i, 0)),
        out_shape=jax.ShapeDtypeStruct((_N, 3), jnp.float32),
    )(rhs3, inputs)

    d0 = dm3[:, 0].reshape(_ROWS, _COLS)
    d1 = dm3[:, 1].reshape(_ROWS, _COLS)
    sq = dm3[:, 2].reshape(_ROWS, _COLS)

    out = pl.pallas_call(
        _loss_body,
        out_shape=jax.ShapeDtypeStruct((1, 1), jnp.float32),
    )(p2t, lg, mk, g_pos, g_neg, d0, d1, sq)
    return out.reshape(())


# speedup vs baseline: 1.5405x; 1.5405x over previous
import jax
import jax.numpy as jnp
from jax import lax
from jax.experimental import pallas as pl
from jax.experimental.pallas import tpu as pltpu

_N = 16384
_C = 512
_K = 1024
_BR = 2048
_NB = _N // _BR
_RPB = _BR // 1024
_ROWS = 16
_COLS = _N // _ROWS
_EPS = 1e-8
_HI = lax.Precision.HIGHEST


def _sortable_u32(x):
    u = lax.bitcast_convert_type(x, jnp.uint32)
    sign = (u >> jnp.uint32(31)).astype(jnp.uint32)
    flip = jnp.where(sign == jnp.uint32(1),
                     jnp.uint32(0xFFFFFFFF), jnp.uint32(0x80000000))
    return u ^ flip


def _kth_threshold(keys_p, keys_n, k):
    kf = jnp.float32(k)

    def body(_, carry):
        lo_p, hi_p, lo_n, hi_n = carry
        mid_p = lo_p + ((hi_p - lo_p) >> jnp.uint32(1)) + ((hi_p - lo_p) & jnp.uint32(1))
        mid_n = lo_n + ((hi_n - lo_n) >> jnp.uint32(1)) + ((hi_n - lo_n) & jnp.uint32(1))
        cnt_p = jnp.sum(jnp.where(keys_p >= mid_p, 1.0, 0.0))
        cnt_n = jnp.sum(jnp.where(keys_n >= mid_n, 1.0, 0.0))
        ok_p = cnt_p >= kf
        ok_n = cnt_n >= kf
        lo_p = jnp.where(ok_p, mid_p, lo_p)
        hi_p = jnp.where(ok_p, hi_p, mid_p - jnp.uint32(1))
        lo_n = jnp.where(ok_n, mid_n, lo_n)
        hi_n = jnp.where(ok_n, hi_n, mid_n - jnp.uint32(1))
        return lo_p, hi_p, lo_n, hi_n

    z = jnp.uint32(0)
    m = jnp.uint32(0xFFFFFFFF)
    lo_p, _, lo_n, _ = lax.fori_loop(0, 32, body, (z, m, z, m))
    return lo_p, lo_n


def _flat_ranks(eq_f32):
    ci = lax.broadcasted_iota(jnp.int32, (_COLS, _COLS), 0)
    cj = lax.broadcasted_iota(jnp.int32, (_COLS, _COLS), 1)
    upper = (ci < cj).astype(jnp.float32)
    row_pfx = jnp.dot(eq_f32, upper)
    ri = lax.broadcasted_iota(jnp.int32, (_ROWS, _ROWS), 0)
    rj = lax.broadcasted_iota(jnp.int32, (_ROWS, _ROWS), 1)
    lower = (rj < ri).astype(jnp.float32)
    row_sums = jnp.sum(eq_f32, axis=1, keepdims=True)
    offs = jnp.dot(lower, row_sums, precision=_HI)
    return row_pfx + offs


def _select_mask(keys, thresh, k):
    gt = (keys > thresh).astype(jnp.float32)
    eq = (keys == thresh).astype(jnp.float32)
    cnt_gt = jnp.sum(gt)
    r = jnp.float32(k) - cnt_gt
    ranks = _flat_ranks(eq)
    pick_eq = eq * jnp.where(ranks < r, 1.0, 0.0)
    return gt + pick_eq


def _body(rhs3_ref, lg_ref, mk_ref, gp_ref, gn_ref, x_ref, out_ref,
          d0_s, d1_s, sq_s):
    i = pl.program_id(0)

    xb = x_ref[...].astype(jnp.bfloat16)
    p2tb = rhs3_ref[...][0:_C, 0:2].astype(jnp.bfloat16)
    dm = jnp.dot(xb, p2tb, preferred_element_type=jnp.float32)
    ones_c = jnp.ones((_C, 1), dtype=jnp.bfloat16)
    sq = jnp.dot(xb * xb, ones_c, preferred_element_type=jnp.float32)
    dmT = jnp.transpose(dm)
    sqT = jnp.transpose(sq)
    for j in range(_RPB):
        sl = slice(1024 * j, 1024 * (j + 1))
        d0_s[pl.ds(i * _RPB + j, 1), :] = dmT[0:1, sl]
        d1_s[pl.ds(i * _RPB + j, 1), :] = dmT[1:2, sl]
        sq_s[pl.ds(i * _RPB + j, 1), :] = sqT[0:1, sl]

    @pl.when(i == _NB - 1)
    def _finish():
        lg = lg_ref[...]
        mk = mk_ref[...]
        sar = jnp.abs(mk - lg)
        w3 = sar * sar * sar
        wp = w3 * mk
        wn = w3 * (1.0 - mk)
        neg_inf = jnp.float32(-jnp.inf)
        logw_p = jnp.where(wp > 0, jnp.log(jnp.maximum(wp, 1e-30)), neg_inf)
        logw_n = jnp.where(wn > 0, jnp.log(jnp.maximum(wn, 1e-30)), neg_inf)
        score_p = logw_p + gp_ref[...]
        score_n = logw_n + gn_ref[...]
        keys_p = _sortable_u32(score_p)
        keys_n = _sortable_u32(score_n)
        t_p, t_n = _kth_threshold(keys_p, keys_n, _K)
        sel_p = _select_mask(keys_p, t_p, _K)
        sel_n = _select_mask(keys_n, t_n, _K)

        rhs3 = rhs3_ref[...]
        d0 = d0_s[...]
        d1 = d1_s[...]
        sq_all = sq_s[...]
        na = jnp.maximum(jnp.sqrt(sq_all), _EPS)
        p0 = rhs3[0:_C, 0:1]
        p1 = rhs3[0:_C, 1:2]
        nb0 = jnp.maximum(jnp.sqrt(jnp.sum(p0 * p0)), _EPS)
        nb1 = jnp.maximum(jnp.sqrt(jnp.sum(p1 * p1)), _EPS)
        s0 = d0 / (na * nb0)
        s1 = d1 / (na * nb1)
        m = jnp.maximum(s0, s1)
        lse = m + jnp.log(jnp.exp(s0 - m) + jnp.exp(s1 - m))
        l0 = lse - s0
        l1 = lse - s1

        sum_p = jnp.sum(sel_p * l0)
        sum_n = jnp.sum(sel_n * l1)
        loss = (sum_p + sum_n) / jnp.float32(2 * _K)
        out_ref[...] = jnp.reshape(loss, (1, 1))


def kernel(inputs, logits, protos, mask):
    skey = jax.random.key(42)
    kp, kn = jax.random.split(skey)
    g_pos = jax.random.gumbel(kp, (_N,), jnp.float32).reshape(_ROWS, _COLS)
    g_neg = jax.random.gumbel(kn, (_N,), jnp.float32).reshape(_ROWS, _COLS)
    p2t = jnp.transpose(protos[:, :, 0])
    rhs3 = jnp.zeros((2 * _C, 3), jnp.float32)
    rhs3 = rhs3.at[0:_C, 0:2].set(p2t)
    rhs3 = rhs3.at[_C:2 * _C, 2].set(1.0)
    lg = logits.reshape(_ROWS, _COLS)
    mk = mask.reshape(_ROWS, _COLS)

    out = pl.pallas_call(
        _body,
        grid=(_NB,),
        in_specs=[
            pl.BlockSpec((2 * _C, 3), lambda i: (0, 0)),
            pl.BlockSpec((_ROWS, _COLS), lambda i: (0, 0)),
            pl.BlockSpec((_ROWS, _COLS), lambda i: (0, 0)),
            pl.BlockSpec((_ROWS, _COLS), lambda i: (0, 0)),
            pl.BlockSpec((_ROWS, _COLS), lambda i: (0, 0)),
            pl.BlockSpec((_BR, _C), lambda i: (i, 0)),
        ],
        out_specs=pl.BlockSpec((1, 1), lambda i: (0, 0)),
        out_shape=jax.ShapeDtypeStruct((1, 1), jnp.float32),
        scratch_shapes=[
            pltpu.VMEM((_ROWS, _COLS), jnp.float32),
            pltpu.VMEM((_ROWS, _COLS), jnp.float32),
            pltpu.VMEM((_ROWS, _COLS), jnp.float32),
        ],
    )(rhs3, lg, mk, g_pos, g_neg, inputs)
    return out.reshape(())


# speedup vs baseline: 1.5977x; 1.0371x over previous
import jax
import jax.numpy as jnp
from jax import lax
from jax.experimental import pallas as pl
from jax.experimental.pallas import tpu as pltpu

_N = 16384
_C = 512
_K = 1024
_BR = 4096
_NB = _N // _BR
_RPB = _BR // 1024
_ROWS = 16
_COLS = _N // _ROWS
_EPS = 1e-8
_HI = lax.Precision.HIGHEST


def _sortable_u32(x):
    u = lax.bitcast_convert_type(x, jnp.uint32)
    sign = (u >> jnp.uint32(31)).astype(jnp.uint32)
    flip = jnp.where(sign == jnp.uint32(1),
                     jnp.uint32(0xFFFFFFFF), jnp.uint32(0x80000000))
    return u ^ flip


def _kth_threshold(keys_p, keys_n, k):
    kf = jnp.float32(k)

    def body(_, carry):
        lo_p, hi_p, lo_n, hi_n = carry
        mid_p = lo_p + ((hi_p - lo_p) >> jnp.uint32(1)) + ((hi_p - lo_p) & jnp.uint32(1))
        mid_n = lo_n + ((hi_n - lo_n) >> jnp.uint32(1)) + ((hi_n - lo_n) & jnp.uint32(1))
        cnt_p = jnp.sum(jnp.where(keys_p >= mid_p, 1.0, 0.0))
        cnt_n = jnp.sum(jnp.where(keys_n >= mid_n, 1.0, 0.0))
        ok_p = cnt_p >= kf
        ok_n = cnt_n >= kf
        lo_p = jnp.where(ok_p, mid_p, lo_p)
        hi_p = jnp.where(ok_p, hi_p, mid_p - jnp.uint32(1))
        lo_n = jnp.where(ok_n, mid_n, lo_n)
        hi_n = jnp.where(ok_n, hi_n, mid_n - jnp.uint32(1))
        return lo_p, hi_p, lo_n, hi_n

    z = jnp.uint32(0)
    m = jnp.uint32(0xFFFFFFFF)
    lo_p, _, lo_n, _ = lax.fori_loop(0, 32, body, (z, m, z, m))
    return lo_p, lo_n


def _flat_ranks(eq_f32):
    ci = lax.broadcasted_iota(jnp.int32, (_COLS, _COLS), 0)
    cj = lax.broadcasted_iota(jnp.int32, (_COLS, _COLS), 1)
    upper = (ci < cj).astype(jnp.float32)
    row_pfx = jnp.dot(eq_f32, upper)
    ri = lax.broadcasted_iota(jnp.int32, (_ROWS, _ROWS), 0)
    rj = lax.broadcasted_iota(jnp.int32, (_ROWS, _ROWS), 1)
    lower = (rj < ri).astype(jnp.float32)
    row_sums = jnp.sum(eq_f32, axis=1, keepdims=True)
    offs = jnp.dot(lower, row_sums, precision=_HI)
    return row_pfx + offs


def _select_mask(keys, thresh, k):
    gt = (keys > thresh).astype(jnp.float32)
    eq = (keys == thresh).astype(jnp.float32)
    cnt_gt = jnp.sum(gt)
    r = jnp.float32(k) - cnt_gt
    ranks = _flat_ranks(eq)
    pick_eq = eq * jnp.where(ranks < r, 1.0, 0.0)
    return gt + pick_eq


def _body(rhs3_ref, lg_ref, mk_ref, gp_ref, gn_ref, x_ref, out_ref,
          d0_s, d1_s, sq_s):
    i = pl.program_id(0)

    xb = x_ref[...].astype(jnp.bfloat16)
    p2tb = rhs3_ref[...][0:_C, 0:2].astype(jnp.bfloat16)
    dm = jnp.dot(xb, p2tb, preferred_element_type=jnp.float32)
    ones_c = jnp.ones((_C, 1), dtype=jnp.bfloat16)
    sq = jnp.dot(xb * xb, ones_c, preferred_element_type=jnp.float32)
    dmT = jnp.transpose(dm)
    sqT = jnp.transpose(sq)
    for j in range(_RPB):
        sl = slice(1024 * j, 1024 * (j + 1))
        d0_s[pl.ds(i * _RPB + j, 1), :] = dmT[0:1, sl]
        d1_s[pl.ds(i * _RPB + j, 1), :] = dmT[1:2, sl]
        sq_s[pl.ds(i * _RPB + j, 1), :] = sqT[0:1, sl]

    @pl.when(i == _NB - 1)
    def _finish():
        lg = lg_ref[...]
        mk = mk_ref[...]
        sar = jnp.abs(mk - lg)
        w3 = sar * sar * sar
        wp = w3 * mk
        wn = w3 * (1.0 - mk)
        neg_inf = jnp.float32(-jnp.inf)
        logw_p = jnp.where(wp > 0, jnp.log(jnp.maximum(wp, 1e-30)), neg_inf)
        logw_n = jnp.where(wn > 0, jnp.log(jnp.maximum(wn, 1e-30)), neg_inf)
        score_p = logw_p + gp_ref[...]
        score_n = logw_n + gn_ref[...]
        keys_p = _sortable_u32(score_p)
        keys_n = _sortable_u32(score_n)
        t_p, t_n = _kth_threshold(keys_p, keys_n, _K)
        sel_p = _select_mask(keys_p, t_p, _K)
        sel_n = _select_mask(keys_n, t_n, _K)

        rhs3 = rhs3_ref[...]
        d0 = d0_s[...]
        d1 = d1_s[...]
        sq_all = sq_s[...]
        na = jnp.maximum(jnp.sqrt(sq_all), _EPS)
        p0 = rhs3[0:_C, 0:1]
        p1 = rhs3[0:_C, 1:2]
        nb0 = jnp.maximum(jnp.sqrt(jnp.sum(p0 * p0)), _EPS)
        nb1 = jnp.maximum(jnp.sqrt(jnp.sum(p1 * p1)), _EPS)
        s0 = d0 / (na * nb0)
        s1 = d1 / (na * nb1)
        m = jnp.maximum(s0, s1)
        lse = m + jnp.log(jnp.exp(s0 - m) + jnp.exp(s1 - m))
        l0 = lse - s0
        l1 = lse - s1

        sum_p = jnp.sum(sel_p * l0)
        sum_n = jnp.sum(sel_n * l1)
        loss = (sum_p + sum_n) / jnp.float32(2 * _K)
        out_ref[...] = jnp.reshape(loss, (1, 1))


def kernel(inputs, logits, protos, mask):
    skey = jax.random.key(42)
    kp, kn = jax.random.split(skey)
    g_pos = jax.random.gumbel(kp, (_N,), jnp.float32).reshape(_ROWS, _COLS)
    g_neg = jax.random.gumbel(kn, (_N,), jnp.float32).reshape(_ROWS, _COLS)
    p2t = jnp.transpose(protos[:, :, 0])
    rhs3 = jnp.zeros((2 * _C, 3), jnp.float32)
    rhs3 = rhs3.at[0:_C, 0:2].set(p2t)
    rhs3 = rhs3.at[_C:2 * _C, 2].set(1.0)
    lg = logits.reshape(_ROWS, _COLS)
    mk = mask.reshape(_ROWS, _COLS)

    out = pl.pallas_call(
        _body,
        grid=(_NB,),
        in_specs=[
            pl.BlockSpec((2 * _C, 3), lambda i: (0, 0)),
            pl.BlockSpec((_ROWS, _COLS), lambda i: (0, 0)),
            pl.BlockSpec((_ROWS, _COLS), lambda i: (0, 0)),
            pl.BlockSpec((_ROWS, _COLS), lambda i: (0, 0)),
            pl.BlockSpec((_ROWS, _COLS), lambda i: (0, 0)),
            pl.BlockSpec((_BR, _C), lambda i: (i, 0)),
        ],
        out_specs=pl.BlockSpec((1, 1), lambda i: (0, 0)),
        out_shape=jax.ShapeDtypeStruct((1, 1), jnp.float32),
        scratch_shapes=[
            pltpu.VMEM((_ROWS, _COLS), jnp.float32),
            pltpu.VMEM((_ROWS, _COLS), jnp.float32),
            pltpu.VMEM((_ROWS, _COLS), jnp.float32),
        ],
    )(rhs3, lg, mk, g_pos, g_neg, inputs)
    return out.reshape(())
